# baseline (speedup 1.0000x reference)
import jax
import jax.numpy as jnp
from jax.experimental import pallas as pl
from jax.experimental.pallas import tpu as pltpu

TILE = 2048
NEG = -1e30


def _fused_kernel(x_ref, w_enc_ref, b_enc_ref, v_ref, w_att_ref, onehot_ref,
                  w_cls_ref, b_cls_ref, out_ref, acc_ref, l_ref, m_ref):
    i = pl.program_id(0)
    nsteps = pl.num_programs(0)

    @pl.when(i == 0)
    def _init():
        acc_ref[...] = jnp.zeros_like(acc_ref)
        l_ref[...] = jnp.zeros_like(l_ref)
        m_ref[0, 0] = NEG

    x = x_ref[...].astype(jnp.bfloat16)
    hf = jnp.maximum(
        jnp.dot(x, w_enc_ref[...], preferred_element_type=jnp.float32)
        + b_enc_ref[...], 0.0)
    h = hf.astype(jnp.bfloat16)
    t = jnp.tanh(jnp.dot(h, v_ref[...], preferred_element_type=jnp.float32))
    s = jnp.dot(t.astype(jnp.bfloat16), w_att_ref[...],
                preferred_element_type=jnp.float32)

    m_old = m_ref[0, 0]
    m_new = jnp.maximum(m_old, jnp.max(s))
    alpha = jnp.exp(m_old - m_new)
    e = jnp.exp(s - m_new)
    p = onehot_ref[...] * e
    l_ref[...] = l_ref[...] * alpha + jnp.sum(p, axis=0, keepdims=True)
    acc_ref[...] = acc_ref[...] * alpha + jax.lax.dot_general(
        h, p.astype(jnp.bfloat16), (((0,), (0,)), ((), ())),
        preferred_element_type=jnp.float32)
    m_ref[0, 0] = m_new

    @pl.when(i == nsteps - 1)
    def _fin():
        z = acc_ref[...] / l_ref[...]
        logits = jax.lax.dot_general(
            w_cls_ref[...], z, (((0,), (0,)), ((), ())),
            preferred_element_type=jnp.float32)
        out_ref[...] = jax.nn.sigmoid(logits + b_cls_ref[...])


def kernel(x, bag_sizes, W_enc, b_enc, V, w_att, W_cls, b_cls):
    total, d_in = x.shape
    d_h = W_enc.shape[1]
    d_att = V.shape[1]
    nb = bag_sizes.shape[0] - 1
    nc = W_cls.shape[1]
    bs = bag_sizes.astype(jnp.int32)
    idx = jnp.arange(total, dtype=jnp.int32)[:, None]
    onehot = ((idx >= bs[None, :-1]) & (idx < bs[None, 1:])
              ).astype(jnp.float32)
    grid = total // TILE

    out = pl.pallas_call(
        _fused_kernel,
        grid=(grid,),
        in_specs=[
            pl.BlockSpec((TILE, d_in), lambda i: (i, 0)),
            pl.BlockSpec((d_in, d_h), lambda i: (0, 0)),
            pl.BlockSpec((1, d_h), lambda i: (0, 0)),
            pl.BlockSpec((d_h, d_att), lambda i: (0, 0)),
            pl.BlockSpec((d_att, 1), lambda i: (0, 0)),
            pl.BlockSpec((TILE, nb), lambda i: (i, 0)),
            pl.BlockSpec((d_h, nc), lambda i: (0, 0)),
            pl.BlockSpec((1, nc), lambda i: (0, 0)),
        ],
        out_specs=pl.BlockSpec((nc, nb), lambda i: (0, 0)),
        out_shape=jax.ShapeDtypeStruct((nc, nb), jnp.float32),
        scratch_shapes=[
            pltpu.VMEM((d_h, nb), jnp.float32),
            pltpu.VMEM((1, nb), jnp.float32),
            pltpu.SMEM((1, 1), jnp.float32),
        ],
        compiler_params=pltpu.CompilerParams(
            dimension_semantics=("arbitrary",)),
    )(x, W_enc.astype(jnp.bfloat16), b_enc.reshape(1, d_h),
      V.astype(jnp.bfloat16), w_att.astype(jnp.bfloat16), onehot,
      W_cls, b_cls.reshape(1, nc))
    return out.T


# speedup vs baseline: 1.1308x; 1.1308x over previous
import jax
import jax.numpy as jnp
from jax.experimental import pallas as pl
from jax.experimental.pallas import tpu as pltpu

TILE = 2048
CLIP = 60.0


def _fused_kernel(x_ref, w_enc_ref, b_enc_ref, v_ref, w_att_ref, onehot_ref,
                  w_cls_ref, b_cls_ref, out_ref, acc_ref, l_ref):
    i = pl.program_id(0)
    nsteps = pl.num_programs(0)

    @pl.when(i == 0)
    def _init():
        acc_ref[...] = jnp.zeros_like(acc_ref)
        l_ref[...] = jnp.zeros_like(l_ref)

    x = x_ref[...].astype(jnp.bfloat16)
    hf = jnp.maximum(
        jnp.dot(x, w_enc_ref[...], preferred_element_type=jnp.float32)
        + b_enc_ref[...], 0.0)
    h = hf.astype(jnp.bfloat16)
    t = jnp.tanh(jnp.dot(h, v_ref[...], preferred_element_type=jnp.float32))
    s = jnp.dot(t.astype(jnp.bfloat16), w_att_ref[...],
                preferred_element_type=jnp.float32)

    e = jnp.exp(jnp.clip(s, -CLIP, CLIP))
    p = onehot_ref[...] * e
    l_ref[...] += jnp.sum(p, axis=0, keepdims=True)
    acc_ref[...] += jax.lax.dot_general(
        p.astype(jnp.bfloat16), h, (((0,), (0,)), ((), ())),
        preferred_element_type=jnp.float32)

    @pl.when(i == nsteps - 1)
    def _fin():
        logits = jax.lax.dot_general(
            w_cls_ref[...], acc_ref[...], (((0,), (1,)), ((), ())),
            preferred_element_type=jnp.float32)
        out_ref[...] = jax.nn.sigmoid(logits / l_ref[...] + b_cls_ref[...])


def kernel(x, bag_sizes, W_enc, b_enc, V, w_att, W_cls, b_cls):
    total, d_in = x.shape
    d_h = W_enc.shape[1]
    d_att = V.shape[1]
    nb = bag_sizes.shape[0] - 1
    nc = W_cls.shape[1]
    bs = bag_sizes.astype(jnp.int32)
    idx = jnp.arange(total, dtype=jnp.int32)[:, None]
    onehot = ((idx >= bs[None, :-1]) & (idx < bs[None, 1:])
              ).astype(jnp.float32)
    grid = total // TILE

    out = pl.pallas_call(
        _fused_kernel,
        grid=(grid,),
        in_specs=[
            pl.BlockSpec((TILE, d_in), lambda i: (i, 0)),
            pl.BlockSpec((d_in, d_h), lambda i: (0, 0)),
            pl.BlockSpec((1, d_h), lambda i: (0, 0)),
            pl.BlockSpec((d_h, d_att), lambda i: (0, 0)),
            pl.BlockSpec((d_att, 1), lambda i: (0, 0)),
            pl.BlockSpec((TILE, nb), lambda i: (i, 0)),
            pl.BlockSpec((d_h, nc), lambda i: (0, 0)),
            pl.BlockSpec((1, nc), lambda i: (0, 0)),
        ],
        out_specs=pl.BlockSpec((nc, nb), lambda i: (0, 0)),
        out_shape=jax.ShapeDtypeStruct((nc, nb), jnp.float32),
        scratch_shapes=[
            pltpu.VMEM((nb, d_h), jnp.float32),
            pltpu.VMEM((1, nb), jnp.float32),
        ],
        compiler_params=pltpu.CompilerParams(
            dimension_semantics=("arbitrary",)),
    )(x, W_enc.astype(jnp.bfloat16), b_enc.reshape(1, d_h),
      V.astype(jnp.bfloat16), w_att.astype(jnp.bfloat16), onehot,
      W_cls, b_cls.reshape(1, nc))
    return out.T
